# baseline (speedup 1.0000x reference)
import jax
import jax.numpy as jnp
from jax.experimental import pallas as pl


def _add_kernel(x_ref, pos_ref, o_ref):
    o_ref[...] = x_ref[...] + pos_ref[...]


def kernel(x, pos_table):
    B, S, D = x.shape
    BS = 256
    x2 = x.reshape(B * S, D)
    nS = S // BS
    out = pl.pallas_call(
        _add_kernel,
        grid=(nS, B),
        in_specs=[
            pl.BlockSpec((BS, D), lambda s, b: (b * nS + s, 0)),
            pl.BlockSpec((BS, D), lambda s, b: (s, 0)),
        ],
        out_specs=pl.BlockSpec((BS, D), lambda s, b: (b * nS + s, 0)),
        out_shape=jax.ShapeDtypeStruct((B * S, D), x.dtype),
    )(x2, pos_table)
    return out.reshape(B, S, D)


# speedup vs baseline: 1.4590x; 1.4590x over previous
import jax
import jax.numpy as jnp
from jax.experimental import pallas as pl


def _add_kernel(x_ref, pos_ref, o_ref):
    o_ref[...] = x_ref[...] + pos_ref[...]


def kernel(x, pos_table):
    B, S, D = x.shape
    BS = 1024
    x2 = x.reshape(B * S, D)
    nS = S // BS
    out = pl.pallas_call(
        _add_kernel,
        grid=(nS, B),
        in_specs=[
            pl.BlockSpec((BS, D), lambda s, b: (b * nS + s, 0)),
            pl.BlockSpec((BS, D), lambda s, b: (s, 0)),
        ],
        out_specs=pl.BlockSpec((BS, D), lambda s, b: (b * nS + s, 0)),
        out_shape=jax.ShapeDtypeStruct((B * S, D), x.dtype),
    )(x2, pos_table)
    return out.reshape(B, S, D)
